# Initial kernel scaffold; baseline (speedup 1.0000x reference)
import functools

import jax
import jax.numpy as jnp
from jax.experimental import pallas as pl
from jax.experimental.pallas import tpu as pltpu

_NPOINT = 2048
_R2 = 0.25
_BIG = jnp.int32(1 << 30)


def _density_kernel(xi_ref, yi_ref, zi_ref, xj_ref, yj_ref, zj_ref, dens_ref):
    dx = xj_ref[...] - xi_ref[...]
    dy = yj_ref[...] - yi_ref[...]
    dz = zj_ref[...] - zi_ref[...]
    d2 = dx * dx
    d2 = d2 + dy * dy
    d2 = d2 + dz * dz
    cnt = jnp.sum(jnp.where(d2 < _R2, 1.0, 0.0).astype(jnp.float32),
                  axis=1, keepdims=True)
    j = pl.program_id(2)

    @pl.when(j == 0)
    def _():
        dens_ref[...] = cnt

    @pl.when(j > 0)
    def _():
        dens_ref[...] = dens_ref[...] + cnt


def _density(x, y, z, ci, cj):
    b, n = x.shape
    xi = x[:, None, :]
    yi = y[:, None, :]
    zi = z[:, None, :]
    xj = x[:, :, None]
    yj = y[:, :, None]
    zj = z[:, :, None]
    ni = n // ci
    nj = n // cj
    ispec = pl.BlockSpec((1, 1, ci), lambda s, i, j: (s, 0, i))
    jspec = pl.BlockSpec((1, cj, 1), lambda s, i, j: (s, j, 0))
    ospec = pl.BlockSpec((1, 1, ci), lambda s, i, j: (s, 0, i))
    dens3 = pl.pallas_call(
        _density_kernel,
        grid=(b, ni, nj),
        in_specs=[ispec, ispec, ispec, jspec, jspec, jspec],
        out_specs=ospec,
        out_shape=jax.ShapeDtypeStruct((b, 1, n), jnp.float32),
        compiler_params=pltpu.CompilerParams(
            dimension_semantics=("parallel", "parallel", "arbitrary")),
    )(xi, yi, zi, xj, yj, zj)
    return dens3.reshape(b, n)


def _fps_kernel(npoint, xs_ref, ys_ref, zs_ref, dens_ref, *idx_refs):
    b, s, l = xs_ref.shape
    xs = xs_ref[...]
    ys = ys_ref[...]
    zs = zs_ref[...]
    dens = dens_ref[...]
    maxd = jnp.max(jnp.max(dens, axis=2, keepdims=True), axis=1, keepdims=True)
    gamma = dens / maxd

    ii = (jax.lax.broadcasted_iota(jnp.int32, (b, s, l), 1) * l
          + jax.lax.broadcasted_iota(jnp.int32, (b, s, l), 2))

    for bb in range(b):
        idx_refs[bb][...] = jnp.zeros((npoint, 1), jnp.int32)

    x0 = xs[:, 0:1, 0:1]
    y0 = ys[:, 0:1, 0:1]
    z0 = zs[:, 0:1, 0:1]
    d0 = jnp.abs(xs - x0) + jnp.abs(ys - y0) + jnp.abs(zs - z0)

    def body(t, min_d):
        score = min_d / gamma
        m = jnp.max(jnp.max(score, axis=2, keepdims=True), axis=1,
                    keepdims=True)
        cand = jnp.where(score == m, ii, _BIG)
        nxt = jnp.min(jnp.min(cand, axis=2, keepdims=True), axis=1,
                      keepdims=True)
        sel = ii == nxt
        zero = jnp.float32(0.0)
        xn = jnp.sum(jnp.sum(jnp.where(sel, xs, zero), axis=2, keepdims=True),
                     axis=1, keepdims=True)
        yn = jnp.sum(jnp.sum(jnp.where(sel, ys, zero), axis=2, keepdims=True),
                     axis=1, keepdims=True)
        zn = jnp.sum(jnp.sum(jnp.where(sel, zs, zero), axis=2, keepdims=True),
                     axis=1, keepdims=True)
        d = jnp.abs(xs - xn) + jnp.abs(ys - yn) + jnp.abs(zs - zn)
        for bb in range(b):
            idx_refs[bb][pl.ds(t, 1), :] = nxt[bb]
        return jnp.minimum(min_d, d)

    jax.lax.fori_loop(1, npoint, body, d0)


def _fps(x, y, z, dens, npoint):
    b, n = x.shape
    sub = 8
    xs = x.reshape(b, sub, n // sub)
    ys = y.reshape(b, sub, n // sub)
    zs = z.reshape(b, sub, n // sub)
    ds = dens.reshape(b, sub, n // sub)
    outs = pl.pallas_call(
        functools.partial(_fps_kernel, npoint),
        out_shape=[jax.ShapeDtypeStruct((npoint, 1), jnp.int32)
                   for _ in range(b)],
    )(xs, ys, zs, ds)
    return jnp.stack(outs, axis=0)[..., 0]


def _run(points_xyz, npoint=_NPOINT, ci=512, cj=1024):
    x = points_xyz[:, :, 0]
    y = points_xyz[:, :, 1]
    z = points_xyz[:, :, 2]
    dens = _density(x, y, z, ci, cj)
    return _fps(x, y, z, dens, npoint)


def kernel(points_xyz, features):
    del features
    return _run(points_xyz)


# speedup vs baseline: 11.8464x; 11.8464x over previous
import functools

import jax
import jax.numpy as jnp
from jax.experimental import pallas as pl
from jax.experimental.pallas import tpu as pltpu

_NPOINT = 2048
_R2 = 0.25
_BIG = 1 << 30


def _density_kernel(xi_ref, yi_ref, zi_ref, xj_ref, yj_ref, zj_ref, dens_ref):
    dx = xj_ref[...] - xi_ref[...]
    dy = yj_ref[...] - yi_ref[...]
    dz = zj_ref[...] - zi_ref[...]
    d2 = dx * dx
    d2 = d2 + dy * dy
    d2 = d2 + dz * dz
    cnt = jnp.sum(jnp.where(d2 < _R2, 1.0, 0.0).astype(jnp.float32),
                  axis=1, keepdims=True)
    j = pl.program_id(2)

    @pl.when(j == 0)
    def _():
        dens_ref[...] = cnt

    @pl.when(j > 0)
    def _():
        dens_ref[...] = dens_ref[...] + cnt


def _density(x, y, z, ci, cj):
    b, n = x.shape
    xi = x[:, None, :]
    yi = y[:, None, :]
    zi = z[:, None, :]
    xj = x[:, :, None]
    yj = y[:, :, None]
    zj = z[:, :, None]
    ni = n // ci
    nj = n // cj
    ispec = pl.BlockSpec((1, 1, ci), lambda s, i, j: (s, 0, i))
    jspec = pl.BlockSpec((1, cj, 1), lambda s, i, j: (s, j, 0))
    ospec = pl.BlockSpec((1, 1, ci), lambda s, i, j: (s, 0, i))
    dens3 = pl.pallas_call(
        _density_kernel,
        grid=(b, ni, nj),
        in_specs=[ispec, ispec, ispec, jspec, jspec, jspec],
        out_specs=ospec,
        out_shape=jax.ShapeDtypeStruct((b, 1, n), jnp.float32),
        compiler_params=pltpu.CompilerParams(
            dimension_semantics=("parallel", "parallel", "arbitrary")),
    )(xi, yi, zi, xj, yj, zj)
    return dens3.reshape(b, n)


def _fps_kernel(npoint, xs_ref, ys_ref, zs_ref, dens_ref, *idx_refs):
    b, s, l = xs_ref.shape
    xs = xs_ref[...]
    ys = ys_ref[...]
    zs = zs_ref[...]
    dens = dens_ref[...]
    maxd = jnp.max(jnp.max(dens, axis=2, keepdims=True), axis=1, keepdims=True)
    gamma = dens / maxd

    ii = (jax.lax.broadcasted_iota(jnp.int32, (b, s, l), 1) * l
          + jax.lax.broadcasted_iota(jnp.int32, (b, s, l), 2))

    for bb in range(b):
        idx_refs[bb][...] = jnp.zeros((npoint, 1), jnp.int32)

    x0 = xs[:, 0:1, 0:1]
    y0 = ys[:, 0:1, 0:1]
    z0 = zs[:, 0:1, 0:1]
    d0 = jnp.abs(xs - x0) + jnp.abs(ys - y0) + jnp.abs(zs - z0)

    def body(t, min_d):
        score = min_d / gamma
        m = jnp.max(jnp.max(score, axis=2, keepdims=True), axis=1,
                    keepdims=True)
        cand = jnp.where(score == m, ii, _BIG)
        nxt = jnp.min(jnp.min(cand, axis=2, keepdims=True), axis=1,
                      keepdims=True)
        sel = ii == nxt
        zero = jnp.float32(0.0)
        xn = jnp.sum(jnp.sum(jnp.where(sel, xs, zero), axis=2, keepdims=True),
                     axis=1, keepdims=True)
        yn = jnp.sum(jnp.sum(jnp.where(sel, ys, zero), axis=2, keepdims=True),
                     axis=1, keepdims=True)
        zn = jnp.sum(jnp.sum(jnp.where(sel, zs, zero), axis=2, keepdims=True),
                     axis=1, keepdims=True)
        d = jnp.abs(xs - xn) + jnp.abs(ys - yn) + jnp.abs(zs - zn)
        for bb in range(b):
            idx_refs[bb][pl.ds(t, 1), :] = nxt[bb]
        return jnp.minimum(min_d, d)

    jax.lax.fori_loop(1, npoint, body, d0)


def _fps(x, y, z, dens, npoint):
    b, n = x.shape
    sub = 8
    xs = x.reshape(b, sub, n // sub)
    ys = y.reshape(b, sub, n // sub)
    zs = z.reshape(b, sub, n // sub)
    ds = dens.reshape(b, sub, n // sub)
    outs = pl.pallas_call(
        functools.partial(_fps_kernel, npoint),
        out_shape=[jax.ShapeDtypeStruct((npoint, 1), jnp.int32)
                   for _ in range(b)],
    )(xs, ys, zs, ds)
    return jnp.stack(outs, axis=0)[..., 0]


def _run(points_xyz, npoint=_NPOINT, ci=512, cj=1024):
    x = points_xyz[:, :, 0]
    y = points_xyz[:, :, 1]
    z = points_xyz[:, :, 2]
    dens = _density(x, y, z, ci, cj)
    return _fps(x, y, z, dens, npoint)


def kernel(points_xyz, features):
    del features
    return _run(points_xyz)
